# Initial kernel scaffold; baseline (speedup 1.0000x reference)
import math

import jax
import jax.numpy as jnp
from jax import lax
from jax.experimental import pallas as pl
from jax.experimental.pallas import tpu as pltpu
from jax.experimental.pallas import tpu_sc as plsc

_N = 10000
_E = 320000
_D = 128
_H = 64
_B = 16
_EPS = 1e-5
_BN_C = 1.0 / math.sqrt(1.0 + _EPS)

_NC = 2
_NS = 16
_NW = _NC * _NS
_EPW = _E // _NW
_KE = 80
_NCH = _EPW // _KE
_NPAD = 10240
_RPT = _NPAD // _NS

_sc_mesh = plsc.VectorSubcoreMesh(
    core_axis_name="c", subcore_axis_name="s", num_cores=_NC, num_subcores=_NS
)


def _deg_body(dst_hbm, out_hbm, idx_v, ones_v, stage_v, deg_sh):
    cid = lax.axis_index("c")
    sid = lax.axis_index("s")
    wid = cid * _NS + sid

    def _init(i, carry):
        ones_v[pl.ds(i * 16, 16)] = jnp.ones((16,), jnp.float32)
        return carry

    lax.fori_loop(0, _KE // 16, _init, 0)

    def _zero(i, carry):
        stage_v[pl.ds(i * 16, 16)] = jnp.zeros((16,), jnp.float32)
        return carry

    lax.fori_loop(0, _RPT // 16, _zero, 0)
    pltpu.sync_copy(stage_v, deg_sh.at[pl.ds(sid * _RPT, _RPT)])
    plsc.subcore_barrier()

    def _edge(i, carry):
        base = wid * _EPW + i * _KE
        pltpu.sync_copy(dst_hbm.at[pl.ds(base, _KE)], idx_v)
        pltpu.sync_copy(ones_v, deg_sh.at[idx_v], add=True)
        return carry

    lax.fori_loop(0, _NCH, _edge, 0)
    plsc.subcore_barrier()
    pltpu.sync_copy(deg_sh.at[pl.ds(sid * _RPT, _RPT)], stage_v)
    pltpu.sync_copy(stage_v, out_hbm.at[cid, pl.ds(sid * _RPT, _RPT)])


_deg_kernel = pl.kernel(
    _deg_body,
    out_type=jax.ShapeDtypeStruct((_NC, _NPAD), jnp.float32),
    mesh=_sc_mesh,
    scratch_types=[
        pltpu.VMEM((_KE,), jnp.int32),
        pltpu.VMEM((_KE,), jnp.float32),
        pltpu.VMEM((_RPT,), jnp.float32),
        pltpu.VMEM_SHARED((_NPAD,), jnp.float32),
    ],
)


def _agg_body(xs_hbm, src_hbm, dst_hbm, out_hbm, src_v, dst_v, rows_v, stage_v,
              agg_sh, sem):
    cid = lax.axis_index("c")
    sid = lax.axis_index("s")
    wid = cid * _NS + sid

    def _zero(r, carry):
        for c in range(_H // 16):
            stage_v[r, pl.ds(c * 16, 16)] = jnp.zeros((16,), jnp.float32)
        return carry

    lax.fori_loop(0, _RPT, _zero, 0)
    pltpu.sync_copy(stage_v, agg_sh.at[pl.ds(sid * _RPT, _RPT)])
    plsc.subcore_barrier()

    def _edge(i, carry):
        base = wid * _EPW + i * _KE
        pltpu.sync_copy(src_hbm.at[pl.ds(base, _KE)], src_v)
        pltpu.sync_copy(dst_hbm.at[pl.ds(base, _KE)], dst_v)
        pltpu.async_copy(xs_hbm.at[src_v], rows_v, sem).wait()
        pltpu.sync_copy(rows_v, agg_sh.at[dst_v], add=True)
        return carry

    lax.fori_loop(0, _NCH, _edge, 0)
    plsc.subcore_barrier()
    pltpu.sync_copy(agg_sh.at[pl.ds(sid * _RPT, _RPT)], stage_v)
    pltpu.sync_copy(stage_v, out_hbm.at[cid, pl.ds(sid * _RPT, _RPT)])


_agg_kernel = pl.kernel(
    _agg_body,
    out_type=jax.ShapeDtypeStruct((_NC, _NPAD, _H), jnp.float32),
    mesh=_sc_mesh,
    scratch_types=[
        pltpu.VMEM((_KE,), jnp.int32),
        pltpu.VMEM((_KE,), jnp.int32),
        pltpu.VMEM((_KE, _H), jnp.float32),
        pltpu.VMEM((_RPT, _H), jnp.float32),
        pltpu.VMEM_SHARED((_NPAD, _H), jnp.float32),
        pltpu.SemaphoreType.DMA,
    ],
)


def _stage_a_body(x_ref, w0_ref, cnt_ref, xs_ref, y_ref, di_ref):
    cnt = cnt_ref[:, 0:1] + cnt_ref[:, 1:2] + 1.0
    dinv = lax.rsqrt(cnt)
    y = jnp.dot(x_ref[...], w0_ref[...], preferred_element_type=jnp.float32)
    y_ref[...] = y
    xs_ref[...] = y * dinv
    di_ref[...] = dinv


_stage_a = pl.pallas_call(
    _stage_a_body,
    out_shape=[
        jax.ShapeDtypeStruct((_N, _H), jnp.float32),
        jax.ShapeDtypeStruct((_N, _H), jnp.float32),
        jax.ShapeDtypeStruct((_N, 1), jnp.float32),
    ],
)


def _stage_b_body(s_ref, y_ref, di_ref, w1_ref, b0_ref, g0_ref, be0_ref,
                  h0_ref, xs1_ref, y1_ref):
    dinv = di_ref[...]
    s = s_ref[0] + s_ref[1]
    agg = dinv * s + dinv * dinv * y_ref[...] + b0_ref[...]
    h0 = jnp.maximum(agg * (g0_ref[...] * _BN_C) + be0_ref[...], 0.0)
    h0_ref[...] = h0
    y1 = jnp.dot(h0, w1_ref[...], preferred_element_type=jnp.float32)
    y1_ref[...] = y1
    xs1_ref[...] = y1 * dinv


_stage_b = pl.pallas_call(
    _stage_b_body,
    out_shape=[
        jax.ShapeDtypeStruct((_N, _H), jnp.float32),
        jax.ShapeDtypeStruct((_N, _H), jnp.float32),
        jax.ShapeDtypeStruct((_N, _H), jnp.float32),
    ],
)


def _stage_c_body(s_ref, y1_ref, di_ref, h0_ref, batch_ref, b1_ref, g1_ref,
                  be1_ref, wh1_ref, bh1_ref, wh2_ref, bh2_ref, wh3_ref,
                  bh3_ref, out_ref):
    dinv = di_ref[...]
    s = s_ref[0] + s_ref[1]
    agg = dinv * s + dinv * dinv * y1_ref[...] + b1_ref[...]
    hn = jnp.maximum(agg * (g1_ref[...] * _BN_C) + be1_ref[...], 0.0)
    h = h0_ref[...] + hn

    bid = batch_ref[...]
    seg = lax.broadcasted_iota(jnp.int32, (_N, _B), 1)
    onehot = (bid == seg).astype(jnp.float32)
    cnt = jnp.sum(onehot, axis=0, keepdims=True)
    m_norm = onehot / jnp.maximum(cnt, 1.0)
    h_mean = lax.dot_general(m_norm, h, (((0,), (0,)), ((), ())),
                             preferred_element_type=jnp.float32)

    neg_inf = jnp.float32(-jnp.inf)
    maxes = [
        jnp.max(jnp.where(bid == b, h, neg_inf), axis=0, keepdims=True)
        for b in range(_B)
    ]
    h_max = jnp.concatenate(maxes, axis=0)

    hg = jnp.concatenate([h_mean, h_max], axis=1)
    z = jnp.maximum(jnp.dot(hg, wh1_ref[...],
                            preferred_element_type=jnp.float32) + bh1_ref[...],
                    0.0)
    z = jnp.maximum(jnp.dot(z, wh2_ref[...],
                            preferred_element_type=jnp.float32) + bh2_ref[...],
                    0.0)
    z = jnp.dot(z, wh3_ref[...], preferred_element_type=jnp.float32) + bh3_ref[...]
    out_ref[...] = jax.nn.sigmoid(z)


_stage_c = pl.pallas_call(
    _stage_c_body,
    out_shape=jax.ShapeDtypeStruct((_B, 1), jnp.float32),
)


def kernel(x, edge_index, batch, W0, b0, g0, be0, W1, b1, g1, be1,
           Wh1, bh1, Wh2, bh2, Wh3, bh3):
    src = edge_index[0]
    dst = edge_index[1]

    counts = _deg_kernel(dst)
    cnt_t = counts[:, :_N].T

    xs0, y0, dinv = _stage_a(x, W0, cnt_t)
    s0 = _agg_kernel(xs0, src, dst)
    h0, xs1, y1 = _stage_b(
        s0[:, :_N, :], y0, dinv, W1,
        b0.reshape(1, _H), g0.reshape(1, _H), be0.reshape(1, _H),
    )
    s1 = _agg_kernel(xs1, src, dst)
    out = _stage_c(
        s1[:, :_N, :], y1, dinv, h0, batch.reshape(_N, 1),
        b1.reshape(1, _H), g1.reshape(1, _H), be1.reshape(1, _H),
        Wh1, bh1.reshape(1, _H), Wh2, bh2.reshape(1, _H // 2),
        Wh3, bh3.reshape(1, 1),
    )
    return out.reshape(_B)


# speedup vs baseline: 14.0681x; 14.0681x over previous
import functools
import math

import jax
import jax.numpy as jnp
from jax import lax
from jax.experimental import pallas as pl
from jax.experimental.pallas import tpu as pltpu
from jax.experimental.pallas import tpu_sc as plsc

_N = 10000
_E = 320000
_D = 128
_H = 64
_B = 16
_EPS = 1e-5
_BN_C = 1.0 / math.sqrt(1.0 + _EPS)

_NC = 2
_NS = 16
_NW = _NC * _NS
_EPW = _E // _NW
_KE = 80
_NCH = _EPW // _KE
_NPAD = 10240
_RPT = _NPAD // _NS

def _deg_body(dst_hbm, out_hbm, idx_v, ones_v, stage_v, deg_sh):
    cid = lax.axis_index("c")
    sid = lax.axis_index("s")
    wid = cid * _NS + sid

    def _init(i, carry):
        ones_v[pl.ds(i * 16, 16)] = jnp.ones((16,), jnp.float32)
        return carry

    lax.fori_loop(0, _KE // 16, _init, 0)

    def _zero(i, carry):
        stage_v[pl.ds(i * 16, 16)] = jnp.zeros((16,), jnp.float32)
        return carry

    lax.fori_loop(0, _RPT // 16, _zero, 0)
    pltpu.sync_copy(stage_v, deg_sh.at[pl.ds(sid * _RPT, _RPT)])
    plsc.subcore_barrier()

    def _edge(i, carry):
        base = wid * _EPW + i * _KE
        pltpu.sync_copy(dst_hbm.at[pl.ds(base, _KE)], idx_v)
        pltpu.sync_copy(ones_v, deg_sh.at[idx_v], add=True)
        return carry

    lax.fori_loop(0, _NCH, _edge, 0)
    plsc.subcore_barrier()
    pltpu.sync_copy(deg_sh.at[pl.ds(sid * _RPT, _RPT)], stage_v)
    pltpu.sync_copy(stage_v, out_hbm.at[cid, pl.ds(sid * _RPT, _RPT)])


@functools.cache
def _sc_mesh():
    return plsc.VectorSubcoreMesh(
        core_axis_name="c", subcore_axis_name="s",
        num_cores=_NC, num_subcores=_NS,
    )


@functools.cache
def _deg_kernel():
    return pl.kernel(
        _deg_body,
        out_type=jax.ShapeDtypeStruct((_NC, _NPAD), jnp.float32),
        mesh=_sc_mesh(),
        scratch_types=[
            pltpu.VMEM((_KE,), jnp.int32),
            pltpu.VMEM((_KE,), jnp.float32),
            pltpu.VMEM((_RPT,), jnp.float32),
            pltpu.VMEM_SHARED((_NPAD,), jnp.float32),
        ],
    )


def _agg_body(xs_hbm, src_hbm, dst_hbm, out_hbm, src_v, dst_v, rows_v, stage_v,
              agg_sh, sem):
    cid = lax.axis_index("c")
    sid = lax.axis_index("s")
    wid = cid * _NS + sid

    def _zero(r, carry):
        for c in range(_H // 16):
            stage_v[r, pl.ds(c * 16, 16)] = jnp.zeros((16,), jnp.float32)
        return carry

    lax.fori_loop(0, _RPT, _zero, 0)
    pltpu.sync_copy(stage_v, agg_sh.at[pl.ds(sid * _RPT, _RPT)])
    plsc.subcore_barrier()

    def _edge(i, carry):
        base = wid * _EPW + i * _KE
        pltpu.sync_copy(src_hbm.at[pl.ds(base, _KE)], src_v)
        pltpu.sync_copy(dst_hbm.at[pl.ds(base, _KE)], dst_v)
        pltpu.async_copy(xs_hbm.at[src_v], rows_v, sem).wait()
        pltpu.sync_copy(rows_v, agg_sh.at[dst_v], add=True)
        return carry

    lax.fori_loop(0, _NCH, _edge, 0)
    plsc.subcore_barrier()
    pltpu.sync_copy(agg_sh.at[pl.ds(sid * _RPT, _RPT)], stage_v)
    pltpu.sync_copy(stage_v, out_hbm.at[cid, pl.ds(sid * _RPT, _RPT)])


@functools.cache
def _agg_kernel():
    return pl.kernel(
        _agg_body,
        out_type=jax.ShapeDtypeStruct((_NC, _NPAD, _H), jnp.float32),
        mesh=_sc_mesh(),
        scratch_types=[
            pltpu.VMEM((_KE,), jnp.int32),
            pltpu.VMEM((_KE,), jnp.int32),
            pltpu.VMEM((_KE, _H), jnp.float32),
            pltpu.VMEM((_RPT, _H), jnp.float32),
            pltpu.VMEM_SHARED((_NPAD, _H), jnp.float32),
            pltpu.SemaphoreType.DMA,
        ],
        compiler_params=pltpu.CompilerParams(use_tc_tiling_on_sc=False),
    )


def _stage_a_body(x_ref, w0_ref, cnt_ref, xs_ref, y_ref, di_ref):
    cnt = cnt_ref[:, 0:1] + cnt_ref[:, 1:2] + 1.0
    dinv = lax.rsqrt(cnt)
    y = jnp.dot(x_ref[...], w0_ref[...], preferred_element_type=jnp.float32)
    y_ref[...] = y
    xs_ref[...] = y * dinv
    di_ref[...] = dinv


_stage_a = pl.pallas_call(
    _stage_a_body,
    out_shape=[
        jax.ShapeDtypeStruct((_N, _H), jnp.float32),
        jax.ShapeDtypeStruct((_N, _H), jnp.float32),
        jax.ShapeDtypeStruct((_N, 1), jnp.float32),
    ],
)


def _stage_b_body(s_ref, y_ref, di_ref, w1_ref, b0_ref, g0_ref, be0_ref,
                  h0_ref, xs1_ref, y1_ref):
    dinv = di_ref[...]
    s = s_ref[0] + s_ref[1]
    agg = dinv * s + dinv * dinv * y_ref[...] + b0_ref[...]
    h0 = jnp.maximum(agg * (g0_ref[...] * _BN_C) + be0_ref[...], 0.0)
    h0_ref[...] = h0
    y1 = jnp.dot(h0, w1_ref[...], preferred_element_type=jnp.float32)
    y1_ref[...] = y1
    xs1_ref[...] = y1 * dinv


_stage_b = pl.pallas_call(
    _stage_b_body,
    out_shape=[
        jax.ShapeDtypeStruct((_N, _H), jnp.float32),
        jax.ShapeDtypeStruct((_N, _H), jnp.float32),
        jax.ShapeDtypeStruct((_N, _H), jnp.float32),
    ],
)


def _stage_c_body(s_ref, y1_ref, di_ref, h0_ref, batch_ref, b1_ref, g1_ref,
                  be1_ref, wh1_ref, bh1_ref, wh2_ref, bh2_ref, wh3_ref,
                  bh3_ref, out_ref):
    dinv = di_ref[...]
    s = s_ref[0] + s_ref[1]
    agg = dinv * s + dinv * dinv * y1_ref[...] + b1_ref[...]
    hn = jnp.maximum(agg * (g1_ref[...] * _BN_C) + be1_ref[...], 0.0)
    h = h0_ref[...] + hn

    bid = batch_ref[...]
    seg = lax.broadcasted_iota(jnp.int32, (_N, _B), 1)
    onehot = (bid == seg).astype(jnp.float32)
    cnt = jnp.sum(onehot, axis=0, keepdims=True)
    m_norm = onehot / jnp.maximum(cnt, 1.0)
    h_mean = lax.dot_general(m_norm, h, (((0,), (0,)), ((), ())),
                             preferred_element_type=jnp.float32)

    neg_inf = jnp.float32(-jnp.inf)
    maxes = [
        jnp.max(jnp.where(bid == b, h, neg_inf), axis=0, keepdims=True)
        for b in range(_B)
    ]
    h_max = jnp.concatenate(maxes, axis=0)

    hg = jnp.concatenate([h_mean, h_max], axis=1)
    z = jnp.maximum(jnp.dot(hg, wh1_ref[...],
                            preferred_element_type=jnp.float32) + bh1_ref[...],
                    0.0)
    z = jnp.maximum(jnp.dot(z, wh2_ref[...],
                            preferred_element_type=jnp.float32) + bh2_ref[...],
                    0.0)
    z = jnp.dot(z, wh3_ref[...], preferred_element_type=jnp.float32) + bh3_ref[...]
    out_ref[...] = jax.nn.sigmoid(z)


_stage_c = pl.pallas_call(
    _stage_c_body,
    out_shape=jax.ShapeDtypeStruct((_B, 1), jnp.float32),
    compiler_params=pltpu.CompilerParams(vmem_limit_bytes=100 * 1024 * 1024),
)


def kernel(x, edge_index, batch, W0, b0, g0, be0, W1, b1, g1, be1,
           Wh1, bh1, Wh2, bh2, Wh3, bh3):
    src = edge_index[0]
    dst = edge_index[1]

    counts = _deg_kernel()(dst)
    cnt_t = counts[:, :_N].T

    xs0, y0, dinv = _stage_a(x, W0, cnt_t)
    s0 = _agg_kernel()(xs0, src, dst)
    h0, xs1, y1 = _stage_b(
        s0[:, :_N, :], y0, dinv, W1,
        b0.reshape(1, _H), g0.reshape(1, _H), be0.reshape(1, _H),
    )
    s1 = _agg_kernel()(xs1, src, dst)
    out = _stage_c(
        s1[:, :_N, :], y1, dinv, h0, batch.reshape(_N, 1),
        b1.reshape(1, _H), g1.reshape(1, _H), be1.reshape(1, _H),
        Wh1, bh1.reshape(1, _H), Wh2, bh2.reshape(1, _H // 2),
        Wh3, bh3.reshape(1, 1),
    )
    return out.reshape(_B)


# speedup vs baseline: 37.1499x; 2.6407x over previous
import functools
import math

import jax
import jax.numpy as jnp
from jax import lax
from jax.experimental import pallas as pl
from jax.experimental.pallas import tpu as pltpu
from jax.experimental.pallas import tpu_sc as plsc

_N = 10000
_E = 320000
_D = 128
_H = 64
_B = 16
_EPS = 1e-5
_BN_C = 1.0 / math.sqrt(1.0 + _EPS)

_NC = 2
_NS = 16
_NW = _NC * _NS
_KE = 80
_NCH = _E // (_NW * _KE)
_J = 5
_NSC = _NCH // _J
_NPAD = 10240
_RPT = _NPAD // _NS
_RST = 160


@functools.cache
def _sc_mesh():
    return plsc.VectorSubcoreMesh(
        core_axis_name="c", subcore_axis_name="s",
        num_cores=_NC, num_subcores=_NS,
    )


def _deg_body(dst_hbm, out_hbm, idx_v, ones_v, stage_v, deg_sh, sem):
    cid = lax.axis_index("c")
    sid = lax.axis_index("s")
    wid = cid * _NS + sid

    def _init(i, carry):
        ones_v[pl.ds(i * 16, 16)] = jnp.ones((16,), jnp.float32)
        return carry

    lax.fori_loop(0, _KE // 16, _init, 0)

    def _zero(i, carry):
        stage_v[pl.ds(i * 16, 16)] = jnp.zeros((16,), jnp.float32)
        return carry

    lax.fori_loop(0, _RPT // 16, _zero, 0)
    pltpu.sync_copy(stage_v, deg_sh.at[pl.ds(sid * _RPT, _RPT)])
    pltpu.sync_copy(dst_hbm.at[pl.ds(wid * _NCH, _NCH)], idx_v)
    plsc.subcore_barrier()

    def _fire(i, carry):
        pltpu.async_copy(ones_v, deg_sh.at[idx_v.at[i]], sem, add=True)
        return carry

    lax.fori_loop(0, _NCH, _fire, 0)

    def _drain(i, carry):
        pltpu.make_async_copy(ones_v, deg_sh.at[idx_v.at[i]], sem).wait()
        return carry

    lax.fori_loop(0, _NCH, _drain, 0)
    plsc.subcore_barrier()
    pltpu.sync_copy(deg_sh.at[pl.ds(sid * _RPT, _RPT)], stage_v)
    pltpu.sync_copy(stage_v, out_hbm.at[cid, pl.ds(sid * _RPT, _RPT)])


@functools.cache
def _deg_kernel():
    return pl.kernel(
        _deg_body,
        out_type=jax.ShapeDtypeStruct((_NC, _NPAD), jnp.float32),
        mesh=_sc_mesh(),
        scratch_types=[
            pltpu.VMEM((_NCH, _KE), jnp.int32),
            pltpu.VMEM((_KE,), jnp.float32),
            pltpu.VMEM((_RPT,), jnp.float32),
            pltpu.VMEM_SHARED((_NPAD,), jnp.float32),
            pltpu.SemaphoreType.DMA,
        ],
        compiler_params=pltpu.CompilerParams(use_tc_tiling_on_sc=False),
    )


def _agg_body(xs_hbm, src_hbm, dst_hbm, out_hbm, src_v, dst_v, rows_v, stage_v,
              agg_sh, sems):
    cid = lax.axis_index("c")
    sid = lax.axis_index("s")
    wid = cid * _NS + sid

    def _zero(r, carry):
        for c in range(_H // 16):
            stage_v[r, pl.ds(c * 16, 16)] = jnp.zeros((16,), jnp.float32)
        return carry

    lax.fori_loop(0, _RST, _zero, 0)
    for q in range(_RPT // _RST):
        pltpu.sync_copy(stage_v, agg_sh.at[pl.ds(sid * _RPT + q * _RST, _RST)])

    pltpu.sync_copy(src_hbm.at[pl.ds(wid * _NCH, _NCH)], src_v)
    pltpu.sync_copy(dst_hbm.at[pl.ds(wid * _NCH, _NCH)], dst_v)
    plsc.subcore_barrier()

    def _issue(b, s):
        for j in range(_J):
            pltpu.async_copy(xs_hbm.at[src_v.at[s * _J + j]],
                             rows_v.at[b, j], sems.at[b, j])

    def _drain(b, s):
        for j in range(_J):
            pltpu.make_async_copy(xs_hbm.at[src_v.at[s * _J + j]],
                                  rows_v.at[b, j], sems.at[b, j]).wait()
            pltpu.sync_copy(rows_v.at[b, j], agg_sh.at[dst_v.at[s * _J + j]],
                            add=True)

    _issue(0, 0)

    def _pipe(gg, carry):
        s0 = gg * 2
        _issue(1, s0 + 1)
        _drain(0, s0)
        _issue(0, s0 + 2)
        _drain(1, s0 + 1)
        return carry

    lax.fori_loop(0, (_NSC - 1) // 2, _pipe, 0)
    _drain(0, _NSC - 1)
    plsc.subcore_barrier()

    for q in range(_RPT // _RST):
        pltpu.sync_copy(agg_sh.at[pl.ds(sid * _RPT + q * _RST, _RST)], stage_v)
        pltpu.sync_copy(stage_v,
                        out_hbm.at[cid, pl.ds(sid * _RPT + q * _RST, _RST)])


@functools.cache
def _agg_kernel():
    return pl.kernel(
        _agg_body,
        out_type=jax.ShapeDtypeStruct((_NC, _NPAD, _H), jnp.float32),
        mesh=_sc_mesh(),
        scratch_types=[
            pltpu.VMEM((_NCH, _KE), jnp.int32),
            pltpu.VMEM((_NCH, _KE), jnp.int32),
            pltpu.VMEM((2, _J, _KE, _H), jnp.float32),
            pltpu.VMEM((_RST, _H), jnp.float32),
            pltpu.VMEM_SHARED((_NPAD, _H), jnp.float32),
            pltpu.SemaphoreType.DMA((2, _J)),
        ],
        compiler_params=pltpu.CompilerParams(use_tc_tiling_on_sc=False),
    )


def _stage_a_body(x_ref, w0_ref, cnt_ref, xs_ref, y_ref, di_ref):
    cnt = cnt_ref[:, 0:1] + cnt_ref[:, 1:2] + 1.0
    dinv = lax.rsqrt(cnt)
    y = jnp.dot(x_ref[...], w0_ref[...], preferred_element_type=jnp.float32)
    y_ref[...] = y
    xs_ref[...] = y * dinv
    di_ref[...] = dinv


_stage_a = pl.pallas_call(
    _stage_a_body,
    out_shape=[
        jax.ShapeDtypeStruct((_N, _H), jnp.float32),
        jax.ShapeDtypeStruct((_N, _H), jnp.float32),
        jax.ShapeDtypeStruct((_N, 1), jnp.float32),
    ],
)


def _stage_b_body(s_ref, y_ref, di_ref, w1_ref, b0_ref, g0_ref, be0_ref,
                  h0_ref, xs1_ref, y1_ref):
    dinv = di_ref[...]
    s = s_ref[0] + s_ref[1]
    agg = dinv * s + dinv * dinv * y_ref[...] + b0_ref[...]
    h0 = jnp.maximum(agg * (g0_ref[...] * _BN_C) + be0_ref[...], 0.0)
    h0_ref[...] = h0
    y1 = jnp.dot(h0, w1_ref[...], preferred_element_type=jnp.float32)
    y1_ref[...] = y1
    xs1_ref[...] = y1 * dinv


_stage_b = pl.pallas_call(
    _stage_b_body,
    out_shape=[
        jax.ShapeDtypeStruct((_N, _H), jnp.float32),
        jax.ShapeDtypeStruct((_N, _H), jnp.float32),
        jax.ShapeDtypeStruct((_N, _H), jnp.float32),
    ],
)


def _stage_c_body(s_ref, y1_ref, di_ref, h0_ref, batch_ref, b1_ref, g1_ref,
                  be1_ref, wh1_ref, bh1_ref, wh2_ref, bh2_ref, wh3_ref,
                  bh3_ref, out_ref):
    dinv = di_ref[...]
    s = s_ref[0] + s_ref[1]
    agg = dinv * s + dinv * dinv * y1_ref[...] + b1_ref[...]
    hn = jnp.maximum(agg * (g1_ref[...] * _BN_C) + be1_ref[...], 0.0)
    h = h0_ref[...] + hn

    bid = batch_ref[...]
    seg = lax.broadcasted_iota(jnp.int32, (_N, _B), 1)
    onehot = (bid == seg).astype(jnp.float32)
    cnt = jnp.sum(onehot, axis=0, keepdims=True)
    m_norm = onehot / jnp.maximum(cnt, 1.0)
    h_mean = lax.dot_general(m_norm, h, (((0,), (0,)), ((), ())),
                             preferred_element_type=jnp.float32)

    neg_inf = jnp.float32(-jnp.inf)
    maxes = [
        jnp.max(jnp.where(bid == b, h, neg_inf), axis=0, keepdims=True)
        for b in range(_B)
    ]
    h_max = jnp.concatenate(maxes, axis=0)

    hg = jnp.concatenate([h_mean, h_max], axis=1)
    z = jnp.maximum(jnp.dot(hg, wh1_ref[...],
                            preferred_element_type=jnp.float32) + bh1_ref[...],
                    0.0)
    z = jnp.maximum(jnp.dot(z, wh2_ref[...],
                            preferred_element_type=jnp.float32) + bh2_ref[...],
                    0.0)
    z = jnp.dot(z, wh3_ref[...], preferred_element_type=jnp.float32) + bh3_ref[...]
    out_ref[...] = jax.nn.sigmoid(z)


_stage_c = pl.pallas_call(
    _stage_c_body,
    out_shape=jax.ShapeDtypeStruct((_B, 1), jnp.float32),
    compiler_params=pltpu.CompilerParams(vmem_limit_bytes=100 * 1024 * 1024),
)


def kernel(x, edge_index, batch, W0, b0, g0, be0, W1, b1, g1, be1,
           Wh1, bh1, Wh2, bh2, Wh3, bh3):
    src = edge_index[0].reshape(_NW * _NCH, _KE)
    dst = edge_index[1].reshape(_NW * _NCH, _KE)

    counts = _deg_kernel()(dst)
    cnt_t = counts[:, :_N].T

    xs0, y0, dinv = _stage_a(x, W0, cnt_t)
    s0 = _agg_kernel()(xs0, src, dst)
    h0, xs1, y1 = _stage_b(
        s0[:, :_N, :], y0, dinv, W1,
        b0.reshape(1, _H), g0.reshape(1, _H), be0.reshape(1, _H),
    )
    s1 = _agg_kernel()(xs1, src, dst)
    out = _stage_c(
        s1[:, :_N, :], y1, dinv, h0, batch.reshape(_N, 1),
        b1.reshape(1, _H), g1.reshape(1, _H), be1.reshape(1, _H),
        Wh1, bh1.reshape(1, _H), Wh2, bh2.reshape(1, _H // 2),
        Wh3, bh3.reshape(1, 1),
    )
    return out.reshape(_B)
